# baseline (speedup 1.0000x reference)
import functools

import jax
import jax.numpy as jnp
from jax import lax
from jax.experimental import pallas as pl
from jax.experimental.pallas import tpu as pltpu
from jax.experimental.pallas import tpu_sc as plsc

_INFO = plsc.get_sparse_core_info()
_NC = _INFO.num_cores
_NS = _INFO.num_subcores
_NW = _NC * _NS
_L = _INFO.num_lanes

_CHUNK = 128


@functools.lru_cache(maxsize=None)
def _build(batch: int, vocab: int, embed_dim: int):
    b_per_w = batch // _NW
    n_gath = b_per_w * embed_dim
    n_chunk = n_gath // _CHUNK
    n_grp = b_per_w // _L
    mesh = plsc.VectorSubcoreMesh(core_axis_name="c", subcore_axis_name="s")

    @functools.partial(
        pl.kernel,
        mesh=mesh,
        out_type=jax.ShapeDtypeStruct((embed_dim, batch), jnp.float32),
        compiler_params=pltpu.CompilerParams(use_tc_tiling_on_sc=False),
        scratch_types=[
            pltpu.VMEM((b_per_w,), jnp.int32),
            pltpu.VMEM((n_chunk, _CHUNK), jnp.int32),
            pltpu.VMEM((embed_dim, b_per_w), jnp.float32),
            pltpu.SemaphoreType.DMA,
        ],
    )
    def gather_kernel(idx_hbm, flat_hbm, out_hbm, idx_v, fidx_v, gbuf_v, sem):
        wid = lax.axis_index("s") * _NC + lax.axis_index("c")
        pltpu.sync_copy(idx_hbm.at[wid], idx_v)

        def build(g, carry):
            vids = idx_v[pl.ds(g * _L, _L)]
            for k in range(embed_dim):
                pos = k * b_per_w
                row = pos // _CHUNK + g // (_CHUNK // _L)
                fidx_v[row, pl.ds((g % (_CHUNK // _L)) * _L, _L)] = (
                    vids + k * vocab
                )
            return carry

        lax.fori_loop(0, n_grp, build, 0)

        cols_per_row = b_per_w // _CHUNK

        def fire(c, carry):
            pltpu.make_async_copy(
                flat_hbm.at[fidx_v.at[c]],
                gbuf_v.at[c // cols_per_row,
                          pl.ds((c % cols_per_row) * _CHUNK, _CHUNK)],
                sem,
            ).start()
            return carry

        lax.fori_loop(0, n_chunk, fire, 0)

        def drain(c, carry):
            pltpu.make_async_copy(
                flat_hbm.at[pl.ds(0, _CHUNK)],
                gbuf_v.at[c // cols_per_row,
                          pl.ds((c % cols_per_row) * _CHUNK, _CHUNK)],
                sem,
            ).wait()
            return carry

        lax.fori_loop(0, n_chunk, drain, 0)
        pltpu.sync_copy(gbuf_v, out_hbm.at[:, pl.ds(wid * b_per_w, b_per_w)])

    return gather_kernel


def kernel(inputs, in_embed_weight):
    batch, = inputs.shape
    vocab, embed_dim = in_embed_weight.shape
    flat = in_embed_weight.T.reshape(vocab * embed_dim)
    idx = inputs.astype(jnp.int32).reshape(_NW, batch // _NW)
    out_t = _build(batch, vocab, embed_dim)(idx, flat)
    return out_t.T
